# speedup vs baseline: 3.3672x; 3.3672x over previous
import functools

import jax
import jax.numpy as jnp
from jax.experimental import pallas as pl
from jax.experimental.pallas import tpu as pltpu

BLK = 8192


def _body(n_total, t_ref, f_ref, c_ref, out_ref, acc_ref):
    i = pl.program_id(0)
    g = pl.num_programs(0)

    @pl.when(i == 0)
    def _init():
        acc_ref[0] = 0.0
        acc_ref[1] = 0.0
        acc_ref[2] = 0.0

    tf = t_ref[0]
    fb = f_ref[...]
    c0 = c_ref[0:1, :]
    c1 = c_ref[1:2, :]
    csel = jnp.where(tf == 1.0, c1, c0)
    diff = fb - csel
    d = jnp.sqrt(jnp.sum(diff * diff, axis=1, keepdims=True))
    s_all = jnp.sum(d)
    s1 = jnp.sum(d * tf)
    n1 = jnp.sum(tf)
    acc_ref[0] += s_all - s1
    acc_ref[1] += s1
    acc_ref[2] += n1

    @pl.when(i == g - 1)
    def _fin():
        n1t = acc_ref[2]
        n0t = jnp.float32(n_total) - n1t
        s0v = acc_ref[0]
        s1v = acc_ref[1]
        r0 = jnp.where(n0t > 0.0, s0v / n0t, 0.0)
        r1 = jnp.where(n1t > 0.0, s1v / n1t, 0.0)
        out_ref[0, 0] = r0 + r1


@jax.jit
def kernel(f, t, center):
    n, d = f.shape
    grid = n // BLK
    t3 = t.astype(jnp.float32).reshape(grid, BLK, 1)
    out = pl.pallas_call(
        functools.partial(_body, n),
        grid=(grid,),
        in_specs=[
            pl.BlockSpec((1, BLK, 1), lambda i: (i, 0, 0)),
            pl.BlockSpec((BLK, d), lambda i: (i, 0)),
            pl.BlockSpec((2, d), lambda i: (0, 0)),
        ],
        out_specs=pl.BlockSpec(
            (1, 1), lambda i: (0, 0), memory_space=pltpu.SMEM
        ),
        out_shape=jax.ShapeDtypeStruct((1, 1), jnp.float32),
        scratch_shapes=[pltpu.SMEM((4,), jnp.float32)],
    )(t3, f, center)
    return out[0, 0]


# speedup vs baseline: 4.7833x; 1.4206x over previous
import functools

import jax
import jax.numpy as jnp
from jax.experimental import pallas as pl
from jax.experimental.pallas import tpu as pltpu

BLK = 8192


def _body(n_total, t_ref, f_ref, c_ref, out_ref, acc_ref):
    i = pl.program_id(0)
    g = pl.num_programs(0)

    @pl.when(i == 0)
    def _init():
        acc_ref[0] = 0.0
        acc_ref[1] = 0.0
        acc_ref[2] = 0.0

    tf = t_ref[0].T
    fb = f_ref[...]
    c0 = c_ref[0:1, :]
    c1 = c_ref[1:2, :]
    csel = jnp.where(tf == 1.0, c1, c0)
    diff = fb - csel
    d = jnp.sqrt(jnp.sum(diff * diff, axis=1, keepdims=True))
    s_all = jnp.sum(d)
    s1 = jnp.sum(d * tf)
    n1 = jnp.sum(tf)
    acc_ref[0] += s_all - s1
    acc_ref[1] += s1
    acc_ref[2] += n1

    @pl.when(i == g - 1)
    def _fin():
        n1t = acc_ref[2]
        n0t = jnp.float32(n_total) - n1t
        s0v = acc_ref[0]
        s1v = acc_ref[1]
        r0 = jnp.where(n0t > 0.0, s0v / n0t, 0.0)
        r1 = jnp.where(n1t > 0.0, s1v / n1t, 0.0)
        out_ref[0, 0] = r0 + r1


@jax.jit
def kernel(f, t, center):
    n, d = f.shape
    grid = n // BLK
    t3 = t.astype(jnp.float32).reshape(grid, 1, BLK)
    out = pl.pallas_call(
        functools.partial(_body, n),
        grid=(grid,),
        in_specs=[
            pl.BlockSpec((1, 1, BLK), lambda i: (i, 0, 0)),
            pl.BlockSpec((BLK, d), lambda i: (i, 0)),
            pl.BlockSpec((2, d), lambda i: (0, 0)),
        ],
        out_specs=pl.BlockSpec(
            (1, 1), lambda i: (0, 0), memory_space=pltpu.SMEM
        ),
        out_shape=jax.ShapeDtypeStruct((1, 1), jnp.float32),
        scratch_shapes=[pltpu.SMEM((4,), jnp.float32)],
    )(t3, f, center)
    return out[0, 0]
